# Initial kernel scaffold; baseline (speedup 1.0000x reference)
import jax
import jax.numpy as jnp
from jax.experimental import pallas as pl


def kernel(x, edge_index, W1l, b1, W1r, W2l, b2, W2r):
    raise NotImplementedError("write your pallas kernel here")


# speedup vs baseline: 6.6989x; 6.6989x over previous
import jax
import jax.numpy as jnp
from jax import lax
from jax.experimental import pallas as pl
from jax.experimental.pallas import tpu as pltpu
from jax.experimental.pallas import tpu_sc as plsc

NC = 2
NS = 16
NW = NC * NS
LANES = 16
C = 128


def _make_sc_agg(n, np_, d, e):
    nwin = e // C
    full_t = nwin // NW
    rem = nwin - full_t * NW
    zr = 64
    zchunks_per_tile = np_ // zr // NS
    cchunk = 1280

    mesh = plsc.VectorSubcoreMesh(
        core_axis_name="c", subcore_axis_name="s", num_cores=NC, num_subcores=NS
    )
    out_type = [
        jax.ShapeDtypeStruct((NC, np_, d), jnp.float32),
        jax.ShapeDtypeStruct((NC, np_), jnp.float32),
    ]
    scratch = [
        pltpu.VMEM((C,), jnp.int32),
        pltpu.VMEM((C,), jnp.int32),
        pltpu.VMEM((C, d), jnp.float32),
        pltpu.VMEM((zr, d), jnp.float32),
        pltpu.VMEM((C,), jnp.float32),
        pltpu.VMEM((cchunk,), jnp.float32),
        pltpu.VMEM_SHARED((np_, d), jnp.float32),
        pltpu.VMEM_SHARED((np_,), jnp.float32),
        pltpu.SemaphoreType.DMA,
    ]

    def body(x_hbm, ei_hbm, out_hbm, cnt_hbm, *scr):
        src_v, dst_v, rows_v, zrow, ones_v, zcnt, acc, cacc, sem = scr

        core = lax.axis_index("c")
        sub = lax.axis_index("s")
        w = core * NS + sub

        def fill16(i, ref, val):
            ref[pl.ds(i * LANES, LANES)] = jnp.full((LANES,), val, jnp.float32)
            return 0

        lax.fori_loop(0, C // LANES, lambda i, _: fill16(i, ones_v, 1.0), 0)
        lax.fori_loop(0, cchunk // LANES, lambda i, _: fill16(i, zcnt, 0.0), 0)

        def zero_zrow(t, _):
            i = t // (d // LANES)
            j = t % (d // LANES)
            zrow[i, pl.ds(j * LANES, LANES)] = jnp.zeros((LANES,), jnp.float32)
            return 0

        lax.fori_loop(0, zr * (d // LANES), zero_zrow, 0)

        for m in range(zchunks_per_tile):
            k = sub * zchunks_per_tile + m
            pltpu.sync_copy(zrow, acc.at[pl.ds(k * zr, zr)])

        n_cchunks = np_ // cchunk
        for m in range(-(-n_cchunks // NS)):
            k = sub + NS * m

            @pl.when(k < n_cchunks)
            def _():
                pltpu.sync_copy(zcnt, cacc.at[pl.ds(k * cchunk, cchunk)])
        plsc.subcore_barrier()

        def window(t, _):
            off = (w + NW * t) * C
            pltpu.sync_copy(ei_hbm.at[0].at[pl.ds(off, C)], src_v)
            pltpu.sync_copy(ei_hbm.at[1].at[pl.ds(off, C)], dst_v)
            pltpu.async_copy(x_hbm.at[src_v], rows_v, sem).wait()
            pltpu.sync_copy(rows_v, acc.at[dst_v], add=True)
            pltpu.sync_copy(ones_v, cacc.at[dst_v], add=True)
            return 0

        lax.fori_loop(0, full_t, window, 0)
        if rem:
            @pl.when(w < rem)
            def _():
                window(full_t, 0)
        plsc.subcore_barrier()

        for m in range(zchunks_per_tile):
            k = sub * zchunks_per_tile + m
            r0 = k * zr
            pltpu.sync_copy(acc.at[pl.ds(r0, zr)], out_hbm.at[core].at[pl.ds(r0, zr)])

        @pl.when(sub == 0)
        def _():
            pltpu.sync_copy(cacc, cnt_hbm.at[core])

    return pl.kernel(body, out_type=out_type, mesh=mesh, scratch_types=scratch)


def _tc_layer(sums, cnt_t, x, wl_t, b, wr_t):
    n, d = x.shape
    bn = 400
    grid = (n // bn,)

    def body(s_ref, c_ref, x_ref, wl_ref, b_ref, wr_ref, o_ref):
        s = s_ref[0] + s_ref[1]
        c = c_ref[:, 0] + c_ref[:, 1]
        mean = s / jnp.maximum(c, 1.0)[:, None]
        o_ref[...] = (
            jnp.dot(mean, wl_ref[...], preferred_element_type=jnp.float32)
            + b_ref[...]
            + jnp.dot(x_ref[...], wr_ref[...], preferred_element_type=jnp.float32)
        )

    return pl.pallas_call(
        body,
        grid=grid,
        in_specs=[
            pl.BlockSpec((NC, bn, d), lambda i: (0, i, 0)),
            pl.BlockSpec((bn, NC), lambda i: (i, 0)),
            pl.BlockSpec((bn, d), lambda i: (i, 0)),
            pl.BlockSpec((d, d), lambda i: (0, 0)),
            pl.BlockSpec((1, d), lambda i: (0, 0)),
            pl.BlockSpec((d, d), lambda i: (0, 0)),
        ],
        out_specs=pl.BlockSpec((bn, d), lambda i: (i, 0)),
        out_shape=jax.ShapeDtypeStruct((n, d), jnp.float32),
    )(sums, cnt_t, x, wl_t, b, wr_t)


def kernel(x, edge_index, W1l, b1, W1r, W2l, b2, W2r):
    n, d = x.shape
    e = edge_index.shape[1]
    np_ = -(-n // 5120) * 5120

    agg = _make_sc_agg(n, np_, d, e)

    sums1, cnt = agg(x, edge_index)
    cnt_t = cnt.T
    h = _tc_layer(sums1, cnt_t, x, W1l.T, b1.reshape(1, d), W1r.T)
    sums2, _ = agg(h, edge_index)
    out = _tc_layer(sums2, cnt_t, h, W2l.T, b2.reshape(1, d), W2r.T)
    return out


# speedup vs baseline: 8.1974x; 1.2237x over previous
import jax
import jax.numpy as jnp
from jax import lax
from jax.experimental import pallas as pl
from jax.experimental.pallas import tpu as pltpu
from jax.experimental.pallas import tpu_sc as plsc

NC = 2
NS = 16
NW = NC * NS
LANES = 16
C = 128


def _make_sc_agg(n, np_, d, e):
    nwin = e // C
    full_t = nwin // NW
    rem = nwin - full_t * NW
    zr = 64
    zchunks_per_tile = np_ // zr // NS
    cchunk = 1280

    mesh = plsc.VectorSubcoreMesh(
        core_axis_name="c", subcore_axis_name="s", num_cores=NC, num_subcores=NS
    )
    out_type = [
        jax.ShapeDtypeStruct((NC, np_, d), jnp.float32),
        jax.ShapeDtypeStruct((NC, np_), jnp.float32),
    ]
    scratch = [
        pltpu.VMEM((C,), jnp.int32),
        pltpu.VMEM((C,), jnp.int32),
        pltpu.VMEM((C,), jnp.int32),
        pltpu.VMEM((C,), jnp.int32),
        pltpu.VMEM((C, d), jnp.float32),
        pltpu.VMEM((C, d), jnp.float32),
        pltpu.VMEM((C,), jnp.float32),
        pltpu.VMEM((cchunk,), jnp.float32),
        pltpu.VMEM_SHARED((np_, d), jnp.float32),
        pltpu.VMEM_SHARED((np_,), jnp.float32),
        pltpu.SemaphoreType.DMA,
        pltpu.SemaphoreType.DMA,
        pltpu.SemaphoreType.DMA,
        pltpu.SemaphoreType.DMA,
        pltpu.SemaphoreType.DMA,
        pltpu.SemaphoreType.DMA,
    ]

    def body(x_hbm, ei_hbm, out_hbm, cnt_hbm, *scr):
        (src0, src1, dst0, dst1, rows0, rows1, ones_v, zcnt, acc, cacc,
         g0, g1, s0, s1, c0, c1) = scr
        src_v = [src0, src1]
        dst_v = [dst0, dst1]
        rows_v = [rows0, rows1]
        gsem = [g0, g1]
        ssem = [s0, s1]
        csem = [c0, c1]

        core = lax.axis_index("c")
        sub = lax.axis_index("s")
        w = core * NS + sub

        def fill16(i, ref, val):
            ref[pl.ds(i * LANES, LANES)] = jnp.full((LANES,), val, jnp.float32)
            return 0

        lax.fori_loop(0, C // LANES, lambda i, _: fill16(i, ones_v, 1.0), 0)
        lax.fori_loop(0, cchunk // LANES, lambda i, _: fill16(i, zcnt, 0.0), 0)

        def zero_rows0(t, _):
            i = t // (d // LANES)
            j = t % (d // LANES)
            rows0[i, pl.ds(j * LANES, LANES)] = jnp.zeros((LANES,), jnp.float32)
            return 0

        lax.fori_loop(0, C * (d // LANES), zero_rows0, 0)

        for m in range(np_ // C // NS):
            k = sub * (np_ // C // NS) + m
            pltpu.sync_copy(rows0, acc.at[pl.ds(k * C, C)])

        n_cchunks = np_ // cchunk
        for m in range(-(-n_cchunks // NS)):
            k = sub + NS * m

            @pl.when(k < n_cchunks)
            def _():
                pltpu.sync_copy(zcnt, cacc.at[pl.ds(k * cchunk, cchunk)])
        plsc.subcore_barrier()

        last_win = nwin - 1

        def stage_and_fire(t, b):
            off = jnp.minimum(w + NW * t, last_win) * C
            pltpu.sync_copy(ei_hbm.at[0].at[pl.ds(off, C)], src_v[b])
            pltpu.sync_copy(ei_hbm.at[1].at[pl.ds(off, C)], dst_v[b])
            pltpu.async_copy(x_hbm.at[src_v[b]], rows_v[b], gsem[b])

        def wait_scatters(b):
            pltpu.make_async_copy(rows_v[b], acc.at[dst_v[b]], ssem[b]).wait()
            pltpu.make_async_copy(ones_v, cacc.at[dst_v[b]], csem[b]).wait()

        def piece(t, b, guard_first):
            pltpu.make_async_copy(x_hbm.at[src_v[b]], rows_v[b], gsem[b]).wait()
            pltpu.async_copy(rows_v[b], acc.at[dst_v[b]], ssem[b], add=True)
            pltpu.async_copy(ones_v, cacc.at[dst_v[b]], csem[b], add=True)
            if guard_first:
                @pl.when(t > 0)
                def _():
                    wait_scatters(1 - b)
            else:
                wait_scatters(1 - b)
            stage_and_fire(t + 1, 1 - b)

        stage_and_fire(0, 0)

        def pair(t2, _):
            piece(2 * t2, 0, True)
            piece(2 * t2 + 1, 1, False)
            return 0

        lax.fori_loop(0, full_t // 2, pair, 0)
        pltpu.make_async_copy(x_hbm.at[src_v[0]], rows_v[0], gsem[0]).wait()
        wait_scatters(1)

        if rem:
            @pl.when(w < rem)
            def _():
                off = (w + NW * full_t) * C
                pltpu.sync_copy(ei_hbm.at[0].at[pl.ds(off, C)], src_v[0])
                pltpu.sync_copy(ei_hbm.at[1].at[pl.ds(off, C)], dst_v[0])
                pltpu.async_copy(x_hbm.at[src_v[0]], rows_v[0], gsem[0]).wait()
                pltpu.sync_copy(rows_v[0], acc.at[dst_v[0]], add=True)
                pltpu.sync_copy(ones_v, cacc.at[dst_v[0]], add=True)
        plsc.subcore_barrier()

        for m in range(zchunks_per_tile):
            k = sub * zchunks_per_tile + m
            r0 = k * zr
            pltpu.sync_copy(acc.at[pl.ds(r0, zr)], out_hbm.at[core].at[pl.ds(r0, zr)])

        @pl.when(sub == 0)
        def _():
            pltpu.sync_copy(cacc, cnt_hbm.at[core])

    return pl.kernel(body, out_type=out_type, mesh=mesh, scratch_types=scratch)


def _tc_layer(sums, cnt_t, x, wl_t, b, wr_t):
    n, d = x.shape
    bn = 400
    grid = (n // bn,)

    def body(s_ref, c_ref, x_ref, wl_ref, b_ref, wr_ref, o_ref):
        s = s_ref[0] + s_ref[1]
        c = c_ref[:, 0] + c_ref[:, 1]
        mean = s / jnp.maximum(c, 1.0)[:, None]
        o_ref[...] = (
            jnp.dot(mean, wl_ref[...], preferred_element_type=jnp.float32)
            + b_ref[...]
            + jnp.dot(x_ref[...], wr_ref[...], preferred_element_type=jnp.float32)
        )

    return pl.pallas_call(
        body,
        grid=grid,
        in_specs=[
            pl.BlockSpec((NC, bn, d), lambda i: (0, i, 0)),
            pl.BlockSpec((bn, NC), lambda i: (i, 0)),
            pl.BlockSpec((bn, d), lambda i: (i, 0)),
            pl.BlockSpec((d, d), lambda i: (0, 0)),
            pl.BlockSpec((1, d), lambda i: (0, 0)),
            pl.BlockSpec((d, d), lambda i: (0, 0)),
        ],
        out_specs=pl.BlockSpec((bn, d), lambda i: (i, 0)),
        out_shape=jax.ShapeDtypeStruct((n, d), jnp.float32),
    )(sums, cnt_t, x, wl_t, b, wr_t)


def kernel(x, edge_index, W1l, b1, W1r, W2l, b2, W2r):
    n, d = x.shape
    e = edge_index.shape[1]
    np_ = -(-n // 5120) * 5120

    agg = _make_sc_agg(n, np_, d, e)

    sums1, cnt = agg(x, edge_index)
    cnt_t = cnt.T
    h = _tc_layer(sums1, cnt_t, x, W1l.T, b1.reshape(1, d), W1r.T)
    sums2, _ = agg(h, edge_index)
    out = _tc_layer(sums2, cnt_t, h, W2l.T, b2.reshape(1, d), W2r.T)
    return out


# speedup vs baseline: 11.5278x; 1.4063x over previous
import jax
import jax.numpy as jnp
from jax import lax
from jax.experimental import pallas as pl
from jax.experimental.pallas import tpu as pltpu
from jax.experimental.pallas import tpu_sc as plsc

NC = 2
NS = 16
NW = NC * NS
LANES = 16
C = 128


def _make_sc_agg(n, np_, d, e):
    nwin = e // C
    full_t = nwin // NW
    rem = nwin - full_t * NW
    zr = 64
    zchunks_per_tile = np_ // zr // NS
    cchunk = 1280

    mesh = plsc.VectorSubcoreMesh(
        core_axis_name="c", subcore_axis_name="s", num_cores=NC, num_subcores=NS
    )
    out_type = [
        jax.ShapeDtypeStruct((NC, np_, d), jnp.float32),
        jax.ShapeDtypeStruct((NC, np_), jnp.float32),
    ]
    scratch = [
        pltpu.VMEM((C,), jnp.int32),
        pltpu.VMEM((C,), jnp.int32),
        pltpu.VMEM((C,), jnp.int32),
        pltpu.VMEM((C,), jnp.int32),
        pltpu.VMEM((C,), jnp.int32),
        pltpu.VMEM((C,), jnp.int32),
        pltpu.VMEM((C, d), jnp.float32),
        pltpu.VMEM((C, d), jnp.float32),
        pltpu.VMEM((C,), jnp.float32),
        pltpu.VMEM((cchunk,), jnp.float32),
        pltpu.VMEM_SHARED((np_, d), jnp.float32),
        pltpu.VMEM_SHARED((np_,), jnp.float32),
        pltpu.SemaphoreType.DMA,
        pltpu.SemaphoreType.DMA,
        pltpu.SemaphoreType.DMA,
        pltpu.SemaphoreType.DMA,
        pltpu.SemaphoreType.DMA,
        pltpu.SemaphoreType.DMA,
        pltpu.SemaphoreType.DMA,
        pltpu.SemaphoreType.DMA,
    ]

    def body(x_hbm, ei_hbm, out_hbm, cnt_hbm, *scr):
        (src0, src1, d0, d1, d2, d3, rows0, rows1, ones_v, zcnt, acc, cacc,
         g0, g1, s0, s1, c0, c1, t0, t1) = scr
        src_v = [src0, src1]
        dst_v = [d0, d1, d2, d3]
        rows_v = [rows0, rows1]
        gsem = [g0, g1]
        ssem = [s0, s1]
        csem = [c0, c1]
        tsem = [t0, t1]

        core = lax.axis_index("c")
        sub = lax.axis_index("s")
        w = core * NS + sub

        def fill16(i, ref, val):
            ref[pl.ds(i * LANES, LANES)] = jnp.full((LANES,), val, jnp.float32)
            return 0

        lax.fori_loop(0, C // LANES, lambda i, _: fill16(i, ones_v, 1.0), 0)
        lax.fori_loop(0, cchunk // LANES, lambda i, _: fill16(i, zcnt, 0.0), 0)

        def zero_rows0(t, _):
            i = t // (d // LANES)
            j = t % (d // LANES)
            rows0[i, pl.ds(j * LANES, LANES)] = jnp.zeros((LANES,), jnp.float32)
            return 0

        lax.fori_loop(0, C * (d // LANES), zero_rows0, 0)

        for m in range(np_ // C // NS):
            k = sub * (np_ // C // NS) + m
            pltpu.sync_copy(rows0, acc.at[pl.ds(k * C, C)])

        n_cchunks = np_ // cchunk
        for m in range(-(-n_cchunks // NS)):
            k = sub + NS * m

            @pl.when(k < n_cchunks)
            def _():
                pltpu.sync_copy(zcnt, cacc.at[pl.ds(k * cchunk, cchunk)])
        plsc.subcore_barrier()

        last_win = nwin - 1

        def stage(t, b, q):
            off = jnp.minimum(w + NW * t, last_win) * C
            pltpu.async_copy(ei_hbm.at[0].at[pl.ds(off, C)], src_v[b], tsem[b])
            pltpu.async_copy(ei_hbm.at[1].at[pl.ds(off, C)], dst_v[q], tsem[b])

        def wait_stage(t, b, q):
            off = jnp.minimum(w + NW * t, last_win) * C
            pltpu.make_async_copy(ei_hbm.at[0].at[pl.ds(off, C)], src_v[b], tsem[b]).wait()
            pltpu.make_async_copy(ei_hbm.at[1].at[pl.ds(off, C)], dst_v[q], tsem[b]).wait()

        def wait_scatters(b, q):
            pltpu.make_async_copy(rows_v[b], acc.at[dst_v[q]], ssem[b]).wait()
            pltpu.make_async_copy(ones_v, cacc.at[dst_v[q]], csem[b]).wait()

        def piece(t, i, guard_first):
            b, q = i % 2, i % 4
            pltpu.make_async_copy(x_hbm.at[src_v[b]], rows_v[b], gsem[b]).wait()
            pltpu.async_copy(rows_v[b], acc.at[dst_v[q]], ssem[b], add=True)
            pltpu.async_copy(ones_v, cacc.at[dst_v[q]], csem[b], add=True)
            stage(t + 2, b, (q + 2) % 4)

            def tail():
                wait_scatters(1 - b, (q + 3) % 4)
                wait_stage(t + 1, 1 - b, (q + 1) % 4)
                pltpu.async_copy(x_hbm.at[src_v[1 - b]], rows_v[1 - b], gsem[1 - b])

            if guard_first:
                @pl.when(t > 0)
                def _():
                    tail()
            else:
                tail()

        pltpu.sync_copy(ei_hbm.at[0].at[pl.ds(w * C, C)], src_v[0])
        pltpu.sync_copy(ei_hbm.at[1].at[pl.ds(w * C, C)], dst_v[0])
        off1 = (w + NW) * C
        pltpu.sync_copy(ei_hbm.at[0].at[pl.ds(off1, C)], src_v[1])
        pltpu.sync_copy(ei_hbm.at[1].at[pl.ds(off1, C)], dst_v[1])
        pltpu.async_copy(x_hbm.at[src_v[0]], rows_v[0], gsem[0])
        pltpu.make_async_copy(x_hbm.at[src_v[0]], rows_v[0], gsem[0]).wait()
        pltpu.async_copy(rows_v[0], acc.at[dst_v[0]], ssem[0], add=True)
        pltpu.async_copy(ones_v, cacc.at[dst_v[0]], csem[0], add=True)
        stage(2, 0, 2)
        pltpu.async_copy(x_hbm.at[src_v[1]], rows_v[1], gsem[1])

        def quad(t4, _):
            t = 4 * t4
            piece(t + 1, 1, False)
            piece(t + 2, 2, False)
            piece(t + 3, 3, False)
            piece(t + 4, 0, False)
            return 0

        lax.fori_loop(0, (full_t - 2) // 4, quad, 0)
        piece(full_t - 1, 1, False)
        pltpu.make_async_copy(x_hbm.at[src_v[0]], rows_v[0], gsem[0]).wait()
        wait_scatters(1, 1)
        wait_stage(full_t + 1, 1, 3)

        if rem:
            @pl.when(w < rem)
            def _():
                pltpu.sync_copy(rows_v[0], acc.at[dst_v[2]], add=True)
                pltpu.sync_copy(ones_v, cacc.at[dst_v[2]], add=True)
        plsc.subcore_barrier()

        for m in range(zchunks_per_tile):
            k = sub * zchunks_per_tile + m
            r0 = k * zr
            pltpu.sync_copy(acc.at[pl.ds(r0, zr)], out_hbm.at[core].at[pl.ds(r0, zr)])

        @pl.when(sub == 0)
        def _():
            pltpu.sync_copy(cacc, cnt_hbm.at[core])

    return pl.kernel(body, out_type=out_type, mesh=mesh, scratch_types=scratch)


def _tc_layer(sums, cnt_t, x, wl_t, b, wr_t):
    n, d = x.shape
    bn = 400
    grid = (n // bn,)

    def body(s_ref, c_ref, x_ref, wl_ref, b_ref, wr_ref, o_ref):
        s = s_ref[0] + s_ref[1]
        c = c_ref[:, 0] + c_ref[:, 1]
        mean = s / jnp.maximum(c, 1.0)[:, None]
        o_ref[...] = (
            jnp.dot(mean, wl_ref[...], preferred_element_type=jnp.float32)
            + b_ref[...]
            + jnp.dot(x_ref[...], wr_ref[...], preferred_element_type=jnp.float32)
        )

    return pl.pallas_call(
        body,
        grid=grid,
        in_specs=[
            pl.BlockSpec((NC, bn, d), lambda i: (0, i, 0)),
            pl.BlockSpec((bn, NC), lambda i: (i, 0)),
            pl.BlockSpec((bn, d), lambda i: (i, 0)),
            pl.BlockSpec((d, d), lambda i: (0, 0)),
            pl.BlockSpec((1, d), lambda i: (0, 0)),
            pl.BlockSpec((d, d), lambda i: (0, 0)),
        ],
        out_specs=pl.BlockSpec((bn, d), lambda i: (i, 0)),
        out_shape=jax.ShapeDtypeStruct((n, d), jnp.float32),
    )(sums, cnt_t, x, wl_t, b, wr_t)


def kernel(x, edge_index, W1l, b1, W1r, W2l, b2, W2r):
    n, d = x.shape
    e = edge_index.shape[1]
    np_ = -(-n // 5120) * 5120

    agg = _make_sc_agg(n, np_, d, e)

    sums1, cnt = agg(x, edge_index)
    cnt_t = cnt.T
    h = _tc_layer(sums1, cnt_t, x, W1l.T, b1.reshape(1, d), W1r.T)
    sums2, _ = agg(h, edge_index)
    out = _tc_layer(sums2, cnt_t, h, W2l.T, b2.reshape(1, d), W2r.T)
    return out
